# Initial kernel scaffold; baseline (speedup 1.0000x reference)
import jax
import jax.numpy as jnp
from jax.experimental import pallas as pl


def kernel(patch_tokens, class_token, conv_w, conv_b, fc_w, fc_b, lam):
    raise NotImplementedError("write your pallas kernel here")


# speedup vs baseline: 1.1393x; 1.1393x over previous
import jax
import jax.numpy as jnp
from jax.experimental import pallas as pl
from jax.experimental.pallas import tpu as pltpu

_CP = 128


def _csra_body(x_ref, ct_ref, wT_ref, cb_ref, fcT_ref, fb_ref, lam_ref,
               out_ref):
    x = x_ref[0]
    s, d = x.shape
    logits = jnp.dot(x.astype(jnp.bfloat16), wT_ref[...],
                     preferred_element_type=jnp.float32) + cb_ref[...]
    attn = jax.nn.sigmoid(logits)
    rowsum = jnp.sum(x, axis=1, keepdims=True)
    s_attn = jnp.sum(attn * rowsum, axis=0, keepdims=True) * (1.0 / (s * d))
    s_global = jnp.dot(ct_ref[0], fcT_ref[...],
                       preferred_element_type=jnp.float32) + fb_ref[...]
    out_ref[0] = s_global + lam_ref[0, 0] * s_attn


def kernel(patch_tokens, class_token, conv_w, conv_b, fc_w, fc_b, lam):
    b, s, d = patch_tokens.shape
    c = conv_w.shape[0]
    pad = _CP - c
    wT = jnp.pad(conv_w, ((0, pad), (0, 0))).T.astype(jnp.bfloat16)
    fcT = jnp.pad(fc_w, ((0, pad), (0, 0))).T
    cb = jnp.pad(conv_b, (0, pad)).reshape(1, _CP)
    fb = jnp.pad(fc_b, (0, pad)).reshape(1, _CP)
    lam2 = jnp.asarray(lam, jnp.float32).reshape(1, 1)
    ct3 = class_token.reshape(b, 1, d)

    out = pl.pallas_call(
        _csra_body,
        grid=(b,),
        in_specs=[
            pl.BlockSpec((1, s, d), lambda i: (i, 0, 0)),
            pl.BlockSpec((1, 1, d), lambda i: (i, 0, 0)),
            pl.BlockSpec((d, _CP), lambda i: (0, 0)),
            pl.BlockSpec((1, _CP), lambda i: (0, 0)),
            pl.BlockSpec((d, _CP), lambda i: (0, 0)),
            pl.BlockSpec((1, _CP), lambda i: (0, 0)),
            pl.BlockSpec((1, 1), lambda i: (0, 0)),
        ],
        out_specs=pl.BlockSpec((1, 1, _CP), lambda i: (i, 0, 0)),
        out_shape=jax.ShapeDtypeStruct((b, 1, _CP), jnp.float32),
        compiler_params=pltpu.CompilerParams(
            dimension_semantics=("parallel",)),
    )(patch_tokens, ct3, wT, cb, fcT, fb, lam2)
    return out[:, 0, :c]


# speedup vs baseline: 1.7191x; 1.5088x over previous
import jax
import jax.numpy as jnp
from jax.experimental import pallas as pl
from jax.experimental.pallas import tpu as pltpu

_CP = 128


def _csra_body(x_ref, ct_ref, wT_ref, cb_ref, fcT_ref, fb_ref, lam_ref,
               out_ref):
    bb, s, d = x_ref.shape
    x = x_ref[...].reshape(bb * s, d)
    logits = jnp.dot(x.astype(jnp.bfloat16), wT_ref[...],
                     preferred_element_type=jnp.float32) + cb_ref[...]
    attn = jax.nn.sigmoid(logits)
    rowsum = jnp.sum(x, axis=1, keepdims=True)
    s_attn = jnp.sum((attn * rowsum).reshape(bb, s, _CP), axis=1) \
        * (1.0 / (s * d))
    s_global = jnp.dot(ct_ref[:, 0, :], fcT_ref[...],
                       preferred_element_type=jnp.float32) + fb_ref[...]
    out_ref[:, 0, :] = s_global + lam_ref[0, 0] * s_attn


def kernel(patch_tokens, class_token, conv_w, conv_b, fc_w, fc_b, lam):
    b, s, d = patch_tokens.shape
    c = conv_w.shape[0]
    pad = _CP - c
    wT = jnp.pad(conv_w, ((0, pad), (0, 0))).T.astype(jnp.bfloat16)
    fcT = jnp.pad(fc_w, ((0, pad), (0, 0))).T
    cb = jnp.pad(conv_b, (0, pad)).reshape(1, _CP)
    fb = jnp.pad(fc_b, (0, pad)).reshape(1, _CP)
    lam2 = jnp.asarray(lam, jnp.float32).reshape(1, 1)
    ct3 = class_token.reshape(b, 1, d)

    bb = 4
    out = pl.pallas_call(
        _csra_body,
        grid=(b // bb,),
        in_specs=[
            pl.BlockSpec((bb, s, d), lambda i: (i, 0, 0)),
            pl.BlockSpec((bb, 1, d), lambda i: (i, 0, 0)),
            pl.BlockSpec((d, _CP), lambda i: (0, 0)),
            pl.BlockSpec((1, _CP), lambda i: (0, 0)),
            pl.BlockSpec((d, _CP), lambda i: (0, 0)),
            pl.BlockSpec((1, _CP), lambda i: (0, 0)),
            pl.BlockSpec((1, 1), lambda i: (0, 0)),
        ],
        out_specs=pl.BlockSpec((bb, 1, _CP), lambda i: (i, 0, 0)),
        out_shape=jax.ShapeDtypeStruct((b, 1, _CP), jnp.float32),
        compiler_params=pltpu.CompilerParams(
            dimension_semantics=("arbitrary",),
            vmem_limit_bytes=100 * 1024 * 1024),
    )(patch_tokens, ct3, wT, cb, fcT, fb, lam2)
    return out[:, 0, :c]


# speedup vs baseline: 1.8698x; 1.0877x over previous
import jax
import jax.numpy as jnp
from jax import lax
from jax.experimental import pallas as pl
from jax.experimental.pallas import tpu as pltpu

_BB = 8


def _csra_body(x_ref, ct_ref, cw_ref, cb_ref, fw_ref, fb_ref, lam_ref,
               out_ref):
    bb, s, d = x_ref.shape
    i = pl.program_id(0)
    x = x_ref[...].reshape(bb * s, d)
    cw = cw_ref[...].astype(jnp.bfloat16)
    logits = lax.dot_general(
        x.astype(jnp.bfloat16), cw,
        (((1,), (1,)), ((), ())),
        preferred_element_type=jnp.float32) + cb_ref[...]
    attn = jax.nn.sigmoid(logits)
    rowsum = jnp.sum(x, axis=1, keepdims=True)
    s_attn = jnp.sum((attn * rowsum).reshape(bb, s, -1), axis=1) \
        * (1.0 / (s * d))
    ct = ct_ref[pl.ds(i * bb, bb), :]
    s_global = lax.dot_general(
        ct, fw_ref[...],
        (((1,), (1,)), ((), ())),
        preferred_element_type=jnp.float32) + fb_ref[...]
    out_ref[...] = s_global + lam_ref[0, 0] * s_attn


def kernel(patch_tokens, class_token, conv_w, conv_b, fc_w, fc_b, lam):
    b, s, d = patch_tokens.shape
    c = conv_w.shape[0]
    cb2 = conv_b.reshape(1, c)
    fb2 = fc_b.reshape(1, c)
    lam2 = jnp.asarray(lam, jnp.float32).reshape(1, 1)

    return pl.pallas_call(
        _csra_body,
        grid=(b // _BB,),
        in_specs=[
            pl.BlockSpec((_BB, s, d), lambda i: (i, 0, 0)),
            pl.BlockSpec((b, d), lambda i: (0, 0)),
            pl.BlockSpec((c, d), lambda i: (0, 0)),
            pl.BlockSpec((1, c), lambda i: (0, 0)),
            pl.BlockSpec((c, d), lambda i: (0, 0)),
            pl.BlockSpec((1, c), lambda i: (0, 0)),
            pl.BlockSpec((1, 1), lambda i: (0, 0)),
        ],
        out_specs=pl.BlockSpec((_BB, c), lambda i: (i, 0)),
        out_shape=jax.ShapeDtypeStruct((b, c), jnp.float32),
        compiler_params=pltpu.CompilerParams(
            dimension_semantics=("arbitrary",),
            vmem_limit_bytes=100 * 1024 * 1024),
    )(patch_tokens, class_token, conv_w, cb2, fc_w, fb2, lam2)


# speedup vs baseline: 1.8792x; 1.0050x over previous
import jax
import jax.numpy as jnp
from jax import lax
from jax.experimental import pallas as pl
from jax.experimental.pallas import tpu as pltpu

_BB = 8


def _csra_body(xa_ref, xb_ref, ct_ref, cw_ref, cb_ref, fw_ref, fb_ref,
               lam_ref, out_ref):
    bb, s, dh = xa_ref.shape
    i = pl.program_id(0)
    xa = xa_ref[...].reshape(bb * s, dh)
    xb = xb_ref[...].reshape(bb * s, dh)
    cw = cw_ref[...].astype(jnp.bfloat16)
    logits = (
        lax.dot_general(xa.astype(jnp.bfloat16), cw[:, :dh],
                        (((1,), (1,)), ((), ())),
                        preferred_element_type=jnp.float32)
        + lax.dot_general(xb.astype(jnp.bfloat16), cw[:, dh:],
                          (((1,), (1,)), ((), ())),
                          preferred_element_type=jnp.float32)
        + cb_ref[...])
    attn = jax.nn.sigmoid(logits)
    rowsum = (jnp.sum(xa, axis=1, keepdims=True)
              + jnp.sum(xb, axis=1, keepdims=True))
    s_attn = jnp.sum((attn * rowsum).reshape(bb, s, -1), axis=1) \
        * (1.0 / (s * 2 * dh))
    ct = ct_ref[pl.ds(i * bb, bb), :]
    s_global = lax.dot_general(
        ct, fw_ref[...],
        (((1,), (1,)), ((), ())),
        preferred_element_type=jnp.float32) + fb_ref[...]
    out_ref[...] = s_global + lam_ref[0, 0] * s_attn


def kernel(patch_tokens, class_token, conv_w, conv_b, fc_w, fc_b, lam):
    b, s, d = patch_tokens.shape
    c = conv_w.shape[0]
    cb2 = conv_b.reshape(1, c)
    fb2 = fc_b.reshape(1, c)
    lam2 = jnp.asarray(lam, jnp.float32).reshape(1, 1)

    return pl.pallas_call(
        _csra_body,
        grid=(b // _BB,),
        in_specs=[
            pl.BlockSpec((_BB, s, d // 2), lambda i: (i, 0, 0)),
            pl.BlockSpec((_BB, s, d // 2), lambda i: (i, 0, 1)),
            pl.BlockSpec((b, d), lambda i: (0, 0)),
            pl.BlockSpec((c, d), lambda i: (0, 0)),
            pl.BlockSpec((1, c), lambda i: (0, 0)),
            pl.BlockSpec((c, d), lambda i: (0, 0)),
            pl.BlockSpec((1, c), lambda i: (0, 0)),
            pl.BlockSpec((1, 1), lambda i: (0, 0)),
        ],
        out_specs=pl.BlockSpec((_BB, c), lambda i: (i, 0)),
        out_shape=jax.ShapeDtypeStruct((b, c), jnp.float32),
        compiler_params=pltpu.CompilerParams(
            dimension_semantics=("arbitrary",),
            vmem_limit_bytes=100 * 1024 * 1024),
    )(patch_tokens, patch_tokens, class_token, conv_w, cb2, fc_w, fb2, lam2)


# speedup vs baseline: 1.8812x; 1.0010x over previous
import jax
import jax.numpy as jnp
from jax import lax
from jax.experimental import pallas as pl
from jax.experimental.pallas import tpu as pltpu

_BB = 8


def _csra_body(xa_ref, xb_ref, ct_ref, cw_ref, cb_ref, fw_ref, fb_ref,
               lam_ref, out_ref):
    bb, s, dh = xa_ref.shape
    c = cw_ref.shape[0]
    i = pl.program_id(0)
    xa = xa_ref[...].reshape(bb * s, dh)
    xb = xb_ref[...].reshape(bb * s, dh)
    cw_aug = jnp.concatenate(
        [cw_ref[...], jnp.ones((1, 2 * dh), jnp.float32)]
    ).astype(jnp.bfloat16)
    raw = (
        lax.dot_general(xa.astype(jnp.bfloat16), cw_aug[:, :dh],
                        (((1,), (1,)), ((), ())),
                        preferred_element_type=jnp.float32)
        + lax.dot_general(xb.astype(jnp.bfloat16), cw_aug[:, dh:],
                          (((1,), (1,)), ((), ())),
                          preferred_element_type=jnp.float32))
    attn = jax.nn.sigmoid(raw[:, :c] + cb_ref[...])
    rowsum = raw[:, c:c + 1]
    s_attn = jnp.sum((attn * rowsum).reshape(bb, s, -1), axis=1) \
        * (1.0 / (s * 2 * dh))
    ct = ct_ref[pl.ds(i * bb, bb), :]
    s_global = lax.dot_general(
        ct, fw_ref[...],
        (((1,), (1,)), ((), ())),
        preferred_element_type=jnp.float32) + fb_ref[...]
    out_ref[...] = s_global + lam_ref[0, 0] * s_attn


def kernel(patch_tokens, class_token, conv_w, conv_b, fc_w, fc_b, lam):
    b, s, d = patch_tokens.shape
    c = conv_w.shape[0]
    cb2 = conv_b.reshape(1, c)
    fb2 = fc_b.reshape(1, c)
    lam2 = jnp.asarray(lam, jnp.float32).reshape(1, 1)

    return pl.pallas_call(
        _csra_body,
        grid=(b // _BB,),
        in_specs=[
            pl.BlockSpec((_BB, s, d // 2), lambda i: (i, 0, 0)),
            pl.BlockSpec((_BB, s, d // 2), lambda i: (i, 0, 1)),
            pl.BlockSpec((b, d), lambda i: (0, 0)),
            pl.BlockSpec((c, d), lambda i: (0, 0)),
            pl.BlockSpec((1, c), lambda i: (0, 0)),
            pl.BlockSpec((c, d), lambda i: (0, 0)),
            pl.BlockSpec((1, c), lambda i: (0, 0)),
            pl.BlockSpec((1, 1), lambda i: (0, 0)),
        ],
        out_specs=pl.BlockSpec((_BB, c), lambda i: (i, 0)),
        out_shape=jax.ShapeDtypeStruct((b, c), jnp.float32),
        compiler_params=pltpu.CompilerParams(
            dimension_semantics=("arbitrary",),
            vmem_limit_bytes=100 * 1024 * 1024),
    )(patch_tokens, patch_tokens, class_token, conv_w, cb2, fc_w, fb2, lam2)


# speedup vs baseline: 1.9677x; 1.0460x over previous
import jax
import jax.numpy as jnp
from jax import lax
from jax.experimental import pallas as pl
from jax.experimental.pallas import tpu as pltpu

_BB = 4


def _csra_body(xa_ref, xb_ref, ct_ref, cw_ref, cb_ref, fw_ref, fb_ref,
               lam_ref, out_ref):
    bb, s, dh = xa_ref.shape
    c = cw_ref.shape[0]
    i = pl.program_id(0)
    xa = xa_ref[...].reshape(bb * s, dh)
    xb = xb_ref[...].reshape(bb * s, dh)
    cw_aug = jnp.concatenate(
        [cw_ref[...], jnp.ones((1, 2 * dh), jnp.float32)]
    ).astype(jnp.bfloat16)
    raw = (
        lax.dot_general(xa.astype(jnp.bfloat16), cw_aug[:, :dh],
                        (((1,), (1,)), ((), ())),
                        preferred_element_type=jnp.float32)
        + lax.dot_general(xb.astype(jnp.bfloat16), cw_aug[:, dh:],
                          (((1,), (1,)), ((), ())),
                          preferred_element_type=jnp.float32))
    attn = jax.nn.sigmoid(raw[:, :c] + cb_ref[...])
    rowsum = raw[:, c:c + 1]
    s_attn = jnp.sum((attn * rowsum).reshape(bb, s, -1), axis=1) \
        * (1.0 / (s * 2 * dh))
    ct = ct_ref[pl.ds((i // 2) * (2 * bb), 2 * bb), :]
    s_global = lax.dot_general(
        ct, fw_ref[...],
        (((1,), (1,)), ((), ())),
        preferred_element_type=jnp.float32) + fb_ref[...]
    s_attn2 = jnp.concatenate([s_attn, s_attn], axis=0)
    rows = lax.broadcasted_iota(jnp.int32, (2 * bb, c), 0)
    mask = (rows // bb) == (i % 2)
    result = s_global + lam_ref[0, 0] * s_attn2
    out_ref[...] = jnp.where(mask, result, out_ref[...])


def kernel(patch_tokens, class_token, conv_w, conv_b, fc_w, fc_b, lam):
    b, s, d = patch_tokens.shape
    c = conv_w.shape[0]
    cb2 = conv_b.reshape(1, c)
    fb2 = fc_b.reshape(1, c)
    lam2 = jnp.asarray(lam, jnp.float32).reshape(1, 1)

    xspec = lambda half: pl.BlockSpec(
        (_BB, s, d // 2), lambda i, _h=half: (i, 0, _h))
    return pl.pallas_call(
        _csra_body,
        grid=(b // _BB,),
        in_specs=[
            xspec(0),
            xspec(1),
            pl.BlockSpec((b, d), lambda i: (0, 0)),
            pl.BlockSpec((c, d), lambda i: (0, 0)),
            pl.BlockSpec((1, c), lambda i: (0, 0)),
            pl.BlockSpec((c, d), lambda i: (0, 0)),
            pl.BlockSpec((1, c), lambda i: (0, 0)),
            pl.BlockSpec((1, 1), lambda i: (0, 0)),
        ],
        out_specs=pl.BlockSpec((2 * _BB, c), lambda i: (i // 2, 0)),
        out_shape=jax.ShapeDtypeStruct((b, c), jnp.float32),
        compiler_params=pltpu.CompilerParams(
            dimension_semantics=("arbitrary",),
            vmem_limit_bytes=100 * 1024 * 1024),
    )(patch_tokens, patch_tokens, class_token, conv_w, cb2, fc_w, fb2, lam2)
